# Initial kernel scaffold; baseline (speedup 1.0000x reference)
import jax
import jax.numpy as jnp
from jax.experimental import pallas as pl


def kernel(x, pe):
    raise NotImplementedError("write your pallas kernel here")


# speedup vs baseline: 2.7672x; 2.7672x over previous
import functools

import jax
import jax.numpy as jnp
from jax import lax
from jax.experimental import pallas as pl
from jax.experimental.pallas import tpu as pltpu
from jax.experimental.pallas import tpu_sc as plsc

D_MODEL = 64
CHUNK = 128


@functools.partial(jax.jit, static_argnums=(2, 3))
def _gather_sc(x2d, pe, n_total, n_chunks_per_worker):
    info = plsc.get_sparse_core_info()
    nc, ns = info.num_cores, info.num_subcores
    nw = nc * ns
    rows_per_worker = n_total // nw

    mesh = plsc.VectorSubcoreMesh(core_axis_name="c", subcore_axis_name="s")

    @functools.partial(
        pl.kernel,
        mesh=mesh,
        out_type=jax.ShapeDtypeStruct((n_total, D_MODEL), jnp.float32),
        scratch_types=[
            pltpu.VMEM((n_chunks_per_worker, CHUNK), jnp.int32),
            pltpu.VMEM((CHUNK, D_MODEL), jnp.float32),
            pltpu.SemaphoreType.DMA,
        ],
        compiler_params=pltpu.CompilerParams(use_tc_tiling_on_sc=False),
    )
    def k(table_hbm, idx_hbm, out_hbm, idx_v, rows_v, sem):
        wid = lax.axis_index("s") * nc + lax.axis_index("c")
        base_chunk = wid * n_chunks_per_worker
        base_row = wid * rows_per_worker
        pltpu.sync_copy(idx_hbm.at[pl.ds(base_chunk, n_chunks_per_worker)], idx_v)

        def body(j, carry):
            pltpu.async_copy(table_hbm.at[idx_v.at[j]], rows_v, sem).wait()
            pltpu.sync_copy(rows_v, out_hbm.at[pl.ds(base_row + j * CHUNK, CHUNK)])
            return carry

        lax.fori_loop(0, n_chunks_per_worker, body, 0)

    return k(pe, x2d)


def kernel(x, pe):
    b, t = x.shape
    n_total = b * t
    info = plsc.get_sparse_core_info()
    nw = info.num_cores * info.num_subcores
    n_chunks_per_worker = n_total // (nw * CHUNK)
    x2d = x.reshape(n_total // CHUNK, CHUNK).astype(jnp.int32)
    out = _gather_sc(x2d, pe, n_total, n_chunks_per_worker)
    return out.reshape(b, t, D_MODEL)


# speedup vs baseline: 2.7977x; 1.0110x over previous
import functools

import jax
import jax.numpy as jnp
from jax import lax
from jax.experimental import pallas as pl
from jax.experimental.pallas import tpu as pltpu
from jax.experimental.pallas import tpu_sc as plsc

D_MODEL = 64
CHUNK = 128
G = 4
NBUF = 2


@functools.partial(jax.jit, static_argnums=(2, 3))
def _gather_sc(x3d, pe, n_total, groups_per_worker):
    info = plsc.get_sparse_core_info()
    nc, ns = info.num_cores, info.num_subcores
    nw = nc * ns
    rows_per_group = G * CHUNK
    rows_per_worker = groups_per_worker * rows_per_group
    assert groups_per_worker % NBUF == 0

    mesh = plsc.VectorSubcoreMesh(core_axis_name="c", subcore_axis_name="s")

    @functools.partial(
        pl.kernel,
        mesh=mesh,
        out_type=jax.ShapeDtypeStruct((n_total, D_MODEL), jnp.float32),
        scratch_types=[
            pltpu.VMEM((groups_per_worker, G, CHUNK), jnp.int32),
            pltpu.VMEM((NBUF, G * CHUNK, D_MODEL), jnp.float32),
            pltpu.SemaphoreType.DMA((NBUF,)),
            pltpu.SemaphoreType.DMA((NBUF,)),
        ],
        compiler_params=pltpu.CompilerParams(use_tc_tiling_on_sc=False),
    )
    def k(table_hbm, idx_hbm, out_hbm, idx_v, rows_v, gsem, osem):
        wid = lax.axis_index("s") * nc + lax.axis_index("c")
        base_group = wid * groups_per_worker
        base_row = wid * rows_per_worker
        pltpu.sync_copy(idx_hbm.at[pl.ds(base_group, groups_per_worker)], idx_v)

        def fire_gathers(g, b):
            for j in range(G):
                pltpu.async_copy(
                    table_hbm.at[idx_v.at[g, j]],
                    rows_v.at[b, pl.ds(j * CHUNK, CHUNK)],
                    gsem.at[b],
                )

        def drain_gathers(g, b):
            for j in range(G):
                pltpu.make_async_copy(
                    table_hbm.at[idx_v.at[g, j]],
                    rows_v.at[b, pl.ds(j * CHUNK, CHUNK)],
                    gsem.at[b],
                ).wait()

        def out_copy(g, b):
            return pltpu.make_async_copy(
                rows_v.at[b],
                out_hbm.at[pl.ds(base_row + g * rows_per_group, rows_per_group)],
                osem.at[b],
            )

        def body(i, carry):
            for b in range(NBUF):
                g = i * NBUF + b

                @pl.when(g >= NBUF)
                def _():
                    out_copy(g - NBUF, b).wait()

                fire_gathers(g, b)
                drain_gathers(g, b)
                out_copy(g, b).start()
            return carry

        lax.fori_loop(0, groups_per_worker // NBUF, body, 0)

        for b in range(NBUF):
            out_copy(groups_per_worker - NBUF + b, b).wait()

    return k(pe, x3d)


def kernel(x, pe):
    b, t = x.shape
    n_total = b * t
    info = plsc.get_sparse_core_info()
    nw = info.num_cores * info.num_subcores
    groups_per_worker = n_total // (nw * G * CHUNK)
    x3d = x.reshape(-1, G, CHUNK).astype(jnp.int32)
    out = _gather_sc(x3d, pe, n_total, groups_per_worker)
    return out.reshape(b, t, D_MODEL)


# speedup vs baseline: 4.9986x; 1.7867x over previous
import functools

import jax
import jax.numpy as jnp
from jax import lax
from jax.experimental import pallas as pl
from jax.experimental.pallas import tpu as pltpu
from jax.experimental.pallas import tpu_sc as plsc

D_MODEL = 64
CHUNK = 128
G = 4
NBUF = 2


@functools.partial(jax.jit, static_argnums=(2, 3))
def _gather_sc(x3d, pe, n_total, groups_per_worker):
    info = plsc.get_sparse_core_info()
    nc, ns = info.num_cores, info.num_subcores
    nw = nc * ns
    rows_per_group = G * CHUNK
    rows_per_worker = groups_per_worker * rows_per_group
    assert groups_per_worker % NBUF == 0

    mesh = plsc.VectorSubcoreMesh(core_axis_name="c", subcore_axis_name="s")

    @functools.partial(
        pl.kernel,
        mesh=mesh,
        out_type=jax.ShapeDtypeStruct((n_total, D_MODEL), jnp.float32),
        scratch_types=[
            pltpu.VMEM((groups_per_worker, G, CHUNK), jnp.int32),
            pltpu.VMEM((NBUF, G * CHUNK, D_MODEL), jnp.float32),
            pltpu.VMEM_SHARED(pe.shape, jnp.float32),
            pltpu.SemaphoreType.DMA((NBUF,)),
            pltpu.SemaphoreType.DMA((NBUF,)),
        ],
        compiler_params=pltpu.CompilerParams(use_tc_tiling_on_sc=False),
    )
    def k(table_hbm, idx_hbm, out_hbm, idx_v, rows_v, table_v, gsem, osem):
        wid = lax.axis_index("s") * nc + lax.axis_index("c")
        base_group = wid * groups_per_worker
        base_row = wid * rows_per_worker
        @pl.when(lax.axis_index("s") == 0)
        def _():
            pltpu.sync_copy(table_hbm, table_v)

        pltpu.sync_copy(idx_hbm.at[pl.ds(base_group, groups_per_worker)], idx_v)
        plsc.subcore_barrier()

        def fire_gathers(g, b):
            for j in range(G):
                pltpu.async_copy(
                    table_v.at[idx_v.at[g, j]],
                    rows_v.at[b, pl.ds(j * CHUNK, CHUNK)],
                    gsem.at[b],
                )

        def drain_gathers(g, b):
            for j in range(G):
                pltpu.make_async_copy(
                    table_v.at[idx_v.at[g, j]],
                    rows_v.at[b, pl.ds(j * CHUNK, CHUNK)],
                    gsem.at[b],
                ).wait()

        def out_copy(g, b):
            return pltpu.make_async_copy(
                rows_v.at[b],
                out_hbm.at[pl.ds(base_row + g * rows_per_group, rows_per_group)],
                osem.at[b],
            )

        def body(i, carry):
            for b in range(NBUF):
                g = i * NBUF + b

                @pl.when(g >= NBUF)
                def _():
                    out_copy(g - NBUF, b).wait()

                fire_gathers(g, b)
                drain_gathers(g, b)
                out_copy(g, b).start()
            return carry

        lax.fori_loop(0, groups_per_worker // NBUF, body, 0)

        for b in range(NBUF):
            out_copy(groups_per_worker - NBUF + b, b).wait()

    return k(pe, x3d)


def kernel(x, pe):
    b, t = x.shape
    n_total = b * t
    info = plsc.get_sparse_core_info()
    nw = info.num_cores * info.num_subcores
    groups_per_worker = n_total // (nw * G * CHUNK)
    x3d = x.reshape(-1, G, CHUNK).astype(jnp.int32)
    out = _gather_sc(x3d, pe, n_total, groups_per_worker)
    return out.reshape(b, t, D_MODEL)


# speedup vs baseline: 5.0068x; 1.0016x over previous
import functools

import jax
import jax.numpy as jnp
from jax import lax
from jax.experimental import pallas as pl
from jax.experimental.pallas import tpu as pltpu
from jax.experimental.pallas import tpu_sc as plsc

D_MODEL = 64
GB = 2
NBUF = 2
SPLITS = ((0, 104), (104, 96))


@jax.jit
def _gather_sc(x, pe):
    bsz, t = x.shape
    info = plsc.get_sparse_core_info()
    nc, ns = info.num_cores, info.num_subcores
    nw = nc * ns
    b_per_w = bsz // nw
    groups_per_worker = b_per_w // GB

    mesh = plsc.VectorSubcoreMesh(core_axis_name="c", subcore_axis_name="s")

    @functools.partial(
        pl.kernel,
        mesh=mesh,
        out_type=jax.ShapeDtypeStruct((bsz, t, D_MODEL), jnp.float32),
        scratch_types=[
            pltpu.VMEM((b_per_w, t), jnp.int32),
            pltpu.VMEM((NBUF, GB, t, D_MODEL), jnp.float32),
            pltpu.VMEM_SHARED(pe.shape, jnp.float32),
            pltpu.SemaphoreType.DMA((NBUF,)),
            pltpu.SemaphoreType.DMA((NBUF,)),
        ],
        compiler_params=pltpu.CompilerParams(use_tc_tiling_on_sc=False),
    )
    def k(table_hbm, idx_hbm, out_hbm, idx_v, rows_v, table_v, gsem, osem):
        wid = lax.axis_index("s") * nc + lax.axis_index("c")
        base_b = wid * b_per_w

        @pl.when(lax.axis_index("s") == 0)
        def _():
            pltpu.sync_copy(table_hbm, table_v)

        pltpu.sync_copy(idx_hbm.at[pl.ds(base_b, b_per_w)], idx_v)
        plsc.subcore_barrier()

        def gather_copies(g, b):
            for i in range(GB):
                for off, ln in SPLITS:
                    yield pltpu.make_async_copy(
                        table_v.at[idx_v.at[g * GB + i, pl.ds(off, ln)]],
                        rows_v.at[b, i, pl.ds(off, ln)],
                        gsem.at[b],
                    )

        def out_copy(g, b):
            return pltpu.make_async_copy(
                rows_v.at[b],
                out_hbm.at[pl.ds(base_b + g * GB, GB)],
                osem.at[b],
            )

        def body(it, carry):
            for b in range(NBUF):
                g = it * NBUF + b

                @pl.when(g >= NBUF)
                def _():
                    out_copy(g - NBUF, b).wait()

                for c in gather_copies(g, b):
                    c.start()
                for c in gather_copies(g, b):
                    c.wait()
                out_copy(g, b).start()
            return carry

        lax.fori_loop(0, groups_per_worker // NBUF, body, 0)

        for b in range(NBUF):
            out_copy(groups_per_worker - NBUF + b, b).wait()

    return k(pe, x.astype(jnp.int32))


def kernel(x, pe):
    return _gather_sc(x, pe)
